# baseline (speedup 1.0000x reference)
import functools

import jax
import jax.numpy as jnp
from jax import lax
from jax.experimental import pallas as pl
from jax.experimental.pallas import tpu as pltpu
from jax.experimental.pallas import tpu_sc as plsc

_NC, _NS = 2, 16
_NW = _NC * _NS
_CHUNK = 128


@functools.partial(jax.jit, static_argnums=())
def _sc_gather(table, idx):
    n, d = idx.shape[0], table.shape[1]
    per_w = n // _NW
    n_chunks = per_w // _CHUNK
    mesh = plsc.VectorSubcoreMesh(core_axis_name="c", subcore_axis_name="s")

    @functools.partial(
        pl.kernel,
        mesh=mesh,
        out_type=jax.ShapeDtypeStruct((n, d), jnp.float32),
        scratch_types=[
            pltpu.VMEM((per_w,), jnp.int32),
            pltpu.VMEM((_CHUNK, d), jnp.float32),
            pltpu.SemaphoreType.DMA,
        ],
    )
    def k(table_hbm, idx_hbm, out_hbm, idx_v, buf, sem):
        wid = lax.axis_index("s") * _NC + lax.axis_index("c")
        base = wid * per_w
        pltpu.sync_copy(idx_hbm.at[pl.ds(base, per_w)], idx_v)

        def chunk(g, carry):
            off = g * _CHUNK
            pltpu.async_copy(
                table_hbm.at[idx_v.at[pl.ds(off, _CHUNK)]], buf, sem
            ).wait()
            pltpu.sync_copy(buf, out_hbm.at[pl.ds(base + off, _CHUNK)])
            return carry

        lax.fori_loop(0, n_chunks, chunk, 0)

    return k(table, idx)


def _head_body(h_ref, wt_ref, b_ref, o_ref):
    bs, hist, _ = o_ref.shape
    w = wt_ref[...]
    bvec = b_ref[...]
    for j in range(bs):
        hj = h_ref[j * hist:(j + 1) * hist, :].astype(jnp.bfloat16)
        o_ref[j, :, :] = (
            jnp.dot(hj, w, preferred_element_type=jnp.float32) + bvec
        )


def _tc_head(h, wt, b2, batch, hist):
    n, d = h.shape
    v = wt.shape[1]
    bs = 16
    return pl.pallas_call(
        _head_body,
        grid=(batch // bs,),
        in_specs=[
            pl.BlockSpec((bs * hist, d), lambda i: (i, 0)),
            pl.BlockSpec((d, v), lambda i: (0, 0)),
            pl.BlockSpec((1, v), lambda i: (0, 0)),
        ],
        out_specs=pl.BlockSpec((bs, hist, v), lambda i: (i, 0, 0)),
        out_shape=jax.ShapeDtypeStruct((batch, hist, v), jnp.float32),
    )(h, wt, b2)


def kernel(x, emb_table, W_head, b_head):
    b, l = x.shape
    v, d = emb_table.shape
    x_flat = x.reshape(-1).astype(jnp.int32)
    table_p = jnp.pad(emb_table, ((0, 0), (0, 128 - d)))
    wt_p = jnp.pad(W_head.T, ((0, 128 - d), (0, 0))).astype(jnp.bfloat16)
    h = _sc_gather(table_p, x_flat)
    return _tc_head(h, wt_p, b_head.reshape(1, v), b, l)


# speedup vs baseline: 1.2420x; 1.2420x over previous
import functools

import jax
import jax.numpy as jnp
from jax import lax
from jax.experimental import pallas as pl
from jax.experimental.pallas import tpu as pltpu
from jax.experimental.pallas import tpu_sc as plsc

_NC, _NS = 2, 16
_NW = _NC * _NS
_CHUNK = 128


def _sc_gather(table, idx):
    n, d = idx.shape[0], table.shape[1]
    per_w = n // _NW
    n_chunks = per_w // _CHUNK
    mesh = plsc.VectorSubcoreMesh(core_axis_name="c", subcore_axis_name="s")

    @functools.partial(
        pl.kernel,
        mesh=mesh,
        out_type=jax.ShapeDtypeStruct((n, d), jnp.float32),
        scratch_types=[
            pltpu.VMEM((per_w,), jnp.int32),
            pltpu.VMEM((_CHUNK, d), jnp.float32),
            pltpu.SemaphoreType.DMA,
        ],
    )
    def k(table_hbm, idx_hbm, out_hbm, idx_v, buf, sem):
        wid = lax.axis_index("s") * _NC + lax.axis_index("c")
        base = wid * per_w
        pltpu.sync_copy(idx_hbm.at[pl.ds(base, per_w)], idx_v)

        def chunk(g, carry):
            off = g * _CHUNK
            pltpu.async_copy(
                table_hbm.at[idx_v.at[pl.ds(off, _CHUNK)]], buf, sem
            ).wait()
            pltpu.sync_copy(buf, out_hbm.at[pl.ds(base + off, _CHUNK)])
            return carry

        lax.fori_loop(0, n_chunks, chunk, 0)

    return k(table, idx)


_BS = 128


def _head_body(h_ref, wt_ref, b_ref, o_ref, acc_ref, sems):
    i = pl.program_id(0)
    n = pl.num_programs(0)
    slot = lax.rem(i, 2)
    bs = _BS
    hist = h_ref.shape[0] // bs
    hb = h_ref[...].astype(jnp.bfloat16)
    acc = jnp.dot(hb, wt_ref[...], preferred_element_type=jnp.float32)
    acc = acc + b_ref[...]

    @pl.when(i >= 2)
    def _():
        pltpu.make_async_copy(
            acc_ref.at[slot], o_ref.at[pl.ds((i - 2) * bs, bs)], sems.at[slot]
        ).wait()

    acc_ref[slot] = acc.reshape(bs, hist, acc.shape[-1])
    pltpu.make_async_copy(
        acc_ref.at[slot], o_ref.at[pl.ds(i * bs, bs)], sems.at[slot]
    ).start()

    @pl.when(i == n - 1)
    def _():
        pltpu.make_async_copy(
            acc_ref.at[slot], o_ref.at[pl.ds(i * bs, bs)], sems.at[slot]
        ).wait()

        @pl.when(n > 1)
        def _():
            other = lax.rem(i + 1, 2)
            pltpu.make_async_copy(
                acc_ref.at[other],
                o_ref.at[pl.ds((i - 1) * bs, bs)],
                sems.at[other],
            ).wait()


def _tc_head(h, wt, b2, batch, hist):
    n, d = h.shape
    v = wt.shape[1]
    return pl.pallas_call(
        _head_body,
        grid=(batch // _BS,),
        in_specs=[
            pl.BlockSpec((_BS * hist, d), lambda i: (i, 0)),
            pl.BlockSpec((d, v), lambda i: (0, 0)),
            pl.BlockSpec((1, v), lambda i: (0, 0)),
        ],
        out_specs=pl.BlockSpec(memory_space=pl.ANY),
        out_shape=jax.ShapeDtypeStruct((batch, hist, v), jnp.float32),
        scratch_shapes=[
            pltpu.VMEM((2, _BS, hist, v), jnp.float32),
            pltpu.SemaphoreType.DMA((2,)),
        ],
    )(h, wt, b2)


def kernel(x, emb_table, W_head, b_head):
    b, l = x.shape
    v, d = emb_table.shape
    x_flat = x.reshape(-1).astype(jnp.int32)
    table_p = jnp.pad(emb_table, ((0, 0), (0, 128 - d)))
    wt_p = jnp.pad(W_head.T, ((0, 128 - d), (0, 0))).astype(jnp.bfloat16)
    h = _sc_gather(table_p, x_flat)
    return _tc_head(h, wt_p, b_head.reshape(1, v), b, l)
